# Initial kernel scaffold; baseline (speedup 1.0000x reference)
import functools

import jax
import jax.numpy as jnp
from jax import lax
from jax.experimental import pallas as pl
from jax.experimental.pallas import tpu as pltpu
from jax.experimental.pallas import tpu_sc as plsc

_NC = 2
_NS = 16
_NW = _NC * _NS


@functools.partial(jax.jit, static_argnums=(2, 3, 4))
def _emb_gather(x_flat, table, B, D, C):
    b_per_w = B // _NW
    n_chunks = b_per_w // C
    mesh = plsc.VectorSubcoreMesh(core_axis_name="c", subcore_axis_name="s")

    @functools.partial(
        pl.kernel,
        out_type=jax.ShapeDtypeStruct((B, D), jnp.float32),
        mesh=mesh,
        scratch_types=[
            pltpu.VMEM((C,), jnp.int32),
            pltpu.VMEM((C, D), jnp.float32),
            pltpu.SemaphoreType.DMA,
        ],
    )
    def k(x_hbm, table_hbm, out_hbm, idx_v, rows_v, sem):
        wid = lax.axis_index("s") * _NC + lax.axis_index("c")
        base = wid * b_per_w

        def body(g, carry):
            off = base + g * C
            pltpu.sync_copy(x_hbm.at[pl.ds(off, C)], idx_v)
            pltpu.async_copy(table_hbm.at[idx_v], rows_v, sem).wait()
            pltpu.sync_copy(rows_v, out_hbm.at[pl.ds(off, C)])
            return carry

        lax.fori_loop(0, n_chunks, body, 0)

    return k(x_flat, table)


def kernel(x, table):
    B = x.shape[0] * x.shape[1]
    D = table.shape[1]
    out = _emb_gather(x.reshape(B).astype(jnp.int32), table, B, D, 1024)
    return out.reshape(x.shape[0], x.shape[1], D)


# speedup vs baseline: 4.8082x; 4.8082x over previous
import functools

import jax
import jax.numpy as jnp
from jax import lax
from jax.experimental import pallas as pl
from jax.experimental.pallas import tpu as pltpu
from jax.experimental.pallas import tpu_sc as plsc

_NC = 2
_NS = 16
_NW = _NC * _NS


@functools.partial(jax.jit, static_argnums=(2, 3, 4))
def _emb_gather(x_flat, table, B, D, C):
    b_per_w = B // _NW
    n_chunks = b_per_w // C
    mesh = plsc.VectorSubcoreMesh(core_axis_name="c", subcore_axis_name="s")

    @functools.partial(
        pl.kernel,
        out_type=jax.ShapeDtypeStruct((B, D), jnp.float32),
        mesh=mesh,
        scratch_types=[
            pltpu.VMEM((C,), jnp.int32),
            pltpu.VMEM((C, D), jnp.float32),
            pltpu.SemaphoreType.DMA,
        ],
        compiler_params=pltpu.CompilerParams(use_tc_tiling_on_sc=False),
    )
    def k(x_hbm, table_hbm, out_hbm, idx_v, rows_v, sem):
        wid = lax.axis_index("s") * _NC + lax.axis_index("c")
        base = wid * b_per_w

        def body(g, carry):
            off = base + g * C
            pltpu.sync_copy(x_hbm.at[pl.ds(off, C)], idx_v)
            pltpu.async_copy(table_hbm.at[idx_v], rows_v, sem).wait()
            pltpu.sync_copy(rows_v, out_hbm.at[pl.ds(off, C)])
            return carry

        lax.fori_loop(0, n_chunks, body, 0)

    return k(x_flat, table)


def kernel(x, table):
    B = x.shape[0] * x.shape[1]
    D = table.shape[1]
    out = _emb_gather(x.reshape(B).astype(jnp.int32), table, B, D, 1024)
    return out.reshape(x.shape[0], x.shape[1], D)


# speedup vs baseline: 5.0477x; 1.0498x over previous
import functools

import jax
import jax.numpy as jnp
from jax import lax
from jax.experimental import pallas as pl
from jax.experimental.pallas import tpu as pltpu
from jax.experimental.pallas import tpu_sc as plsc

_NC = 2
_NS = 16
_NW = _NC * _NS


@functools.partial(jax.jit, static_argnums=(2, 3, 4))
def _emb_gather(x_flat, table, B, D, C):
    b_per_w = B // _NW
    n_chunks = b_per_w // C
    assert n_chunks * C == b_per_w and n_chunks >= 2
    mesh = plsc.VectorSubcoreMesh(core_axis_name="c", subcore_axis_name="s")

    @functools.partial(
        pl.kernel,
        out_type=jax.ShapeDtypeStruct((B, D), jnp.float32),
        mesh=mesh,
        scratch_types=[
            pltpu.VMEM((2, C), jnp.int32),
            pltpu.VMEM((2, C, D), jnp.float32),
            pltpu.SemaphoreType.DMA((2,)),
            pltpu.SemaphoreType.DMA((2,)),
            pltpu.SemaphoreType.DMA((2,)),
        ],
        compiler_params=pltpu.CompilerParams(use_tc_tiling_on_sc=False),
    )
    def k2(x_hbm, table_hbm, out_hbm, idx_v, rows_v, s_idx, s_g, s_st):
        wid = lax.axis_index("s") * _NC + lax.axis_index("c")
        base = wid * b_per_w

        def idx_copy(g, b):
            return pltpu.make_async_copy(
                x_hbm.at[pl.ds(base + g * C, C)], idx_v.at[b], s_idx.at[b])

        def gather_copy(b):
            return pltpu.make_async_copy(
                table_hbm.at[idx_v.at[b]], rows_v.at[b], s_g.at[b])

        def store_copy(g, b):
            return pltpu.make_async_copy(
                rows_v.at[b], out_hbm.at[pl.ds(base + g * C, C)], s_st.at[b])

        idx_copy(0, 0).start()
        idx_copy(1, 1).start()
        idx_copy(0, 0).wait()
        gather_copy(0).start()

        def body(g, carry):
            b = g % 2
            pb = 1 - b
            @pl.when(g >= 2)
            def _():
                store_copy(g, b).wait()
            idx_copy(g, b).wait()
            gather_copy(b).start()
            gather_copy(pb).wait()
            store_copy(g - 1, pb).start()

            @pl.when(g + 1 < n_chunks)
            def _():
                idx_copy(g + 1, pb).start()

            return carry

        lax.fori_loop(1, n_chunks, body, 0, unroll=2)

        lb = (n_chunks - 1) % 2
        gather_copy(lb).wait()
        store_copy(n_chunks - 1, lb).start()
        store_copy(n_chunks - 2, 1 - lb).wait()
        store_copy(n_chunks - 1, lb).wait()

    return k2(x_flat, table)


def kernel(x, table):
    B = x.shape[0] * x.shape[1]
    D = table.shape[1]
    out = _emb_gather(x.reshape(B).astype(jnp.int32), table, B, D, 1024)
    return out.reshape(x.shape[0], x.shape[1], D)
